# baseline (speedup 1.0000x reference)
import functools

import jax
import jax.numpy as jnp
from jax import lax
from jax.experimental import pallas as pl
from jax.experimental.pallas import tpu as pltpu
from jax.experimental.pallas import tpu_sc as plsc

B = 16384
EMB = 64

_info = plsc.get_sparse_core_info()
_NC, _NS = _info.num_cores, _info.num_subcores
_NW = _NC * _NS
_BPW = B // _NW


@functools.partial(
    pl.kernel,
    out_type=(
        jax.ShapeDtypeStruct((B, EMB), jnp.float32),
        jax.ShapeDtypeStruct((B, EMB), jnp.float32),
    ),
    mesh=plsc.VectorSubcoreMesh(core_axis_name="c", subcore_axis_name="s"),
    scratch_types=(
        pltpu.VMEM((_BPW,), jnp.int32),
        pltpu.VMEM((_BPW,), jnp.int32),
        pltpu.VMEM((_BPW, EMB), jnp.float32),
        pltpu.VMEM((_BPW, EMB), jnp.float32),
        pltpu.SemaphoreType.DMA,
        pltpu.SemaphoreType.DMA,
    ),
    compiler_params=pltpu.CompilerParams(use_tc_tiling_on_sc=False),
)
def _sc_gather(user_table, artist_table, user_idx, artist_idx,
               u_out, a_out, uidx_v, aidx_v, urows_v, arows_v, usem, asem):
    wid = lax.axis_index("s") * _NC + lax.axis_index("c")
    base = wid * _BPW
    pltpu.sync_copy(user_idx.at[pl.ds(base, _BPW)], uidx_v)
    pltpu.sync_copy(artist_idx.at[pl.ds(base, _BPW)], aidx_v)
    ucp = pltpu.async_copy(user_table.at[uidx_v], urows_v, usem)
    acp = pltpu.async_copy(artist_table.at[aidx_v], arows_v, asem)
    ucp.wait()
    acp.wait()
    pltpu.sync_copy(urows_v, u_out.at[pl.ds(base, _BPW)])
    pltpu.sync_copy(arows_v, a_out.at[pl.ds(base, _BPW)])


_BLK = 2048


def _mlp_body(u_ref, a_ref, w1u_ref, w1a_ref, b1_ref, w2_ref, b2_ref,
              w3_ref, b3_ref, w4_ref, b4_ref, out_ref):
    u = u_ref[...]
    a = a_ref[...]
    x = (jnp.dot(u, w1u_ref[...], preferred_element_type=jnp.float32)
         + jnp.dot(a, w1a_ref[...], preferred_element_type=jnp.float32)
         + b1_ref[...])
    x = jnp.maximum(x, 0.0)
    x = jnp.dot(x, w2_ref[...], preferred_element_type=jnp.float32) + b2_ref[...]
    x = jnp.maximum(x, 0.0)
    x = jnp.dot(x, w3_ref[...], preferred_element_type=jnp.float32) + b3_ref[...]
    x = jnp.maximum(x, 0.0)
    z = jnp.sum(x * w4_ref[...], axis=1) + b4_ref[0]
    out_ref[...] = 1.0 / (1.0 + jnp.exp(-z))


def _mlp(u_emb, a_emb, W1, b1, W2, b2, W3, b3, W4, b4):
    w1u, w1a = W1[:EMB], W1[EMB:]
    w4 = jnp.reshape(W4, (1, 32))
    grid = (B // _BLK,)
    full = lambda i: (0, 0)
    return pl.pallas_call(
        _mlp_body,
        grid=grid,
        in_specs=[
            pl.BlockSpec((_BLK, EMB), lambda i: (i, 0)),
            pl.BlockSpec((_BLK, EMB), lambda i: (i, 0)),
            pl.BlockSpec((EMB, 128), full),
            pl.BlockSpec((EMB, 128), full),
            pl.BlockSpec((1, 128), full),
            pl.BlockSpec((128, 64), full),
            pl.BlockSpec((1, 64), full),
            pl.BlockSpec((64, 32), full),
            pl.BlockSpec((1, 32), full),
            pl.BlockSpec((1, 32), full),
            pl.BlockSpec((1,), lambda i: (0,)),
        ],
        out_specs=pl.BlockSpec((_BLK,), lambda i: (i,)),
        out_shape=jax.ShapeDtypeStruct((B,), jnp.float32),
    )(u_emb, a_emb, w1u, w1a, b1[None, :], W2, b2[None, :], W3, b3[None, :],
      w4, b4)


def kernel(user_idx, artist_idx, user_table, artist_table,
           W1, b1, W2, b2, W3, b3, W4, b4):
    u_emb, a_emb = _sc_gather(user_table, artist_table,
                              user_idx.astype(jnp.int32),
                              artist_idx.astype(jnp.int32))
    return _mlp(u_emb, a_emb, W1, b1, W2, b2, W3, b3, W4, b4)


# speedup vs baseline: 1.6204x; 1.6204x over previous
import functools

import jax
import jax.numpy as jnp
from jax import lax
from jax.experimental import pallas as pl
from jax.experimental.pallas import tpu as pltpu
from jax.experimental.pallas import tpu_sc as plsc

B = 16384
EMB = 64
N_USERS = 1000000
N_ARTISTS = 100000

_info = plsc.get_sparse_core_info()
_NC, _NS = _info.num_cores, _info.num_subcores
_NW = _NC * _NS
_BPW = B // _NW
_CH = 32
_NCH = _BPW // _CH


@functools.partial(
    pl.kernel,
    out_type=(
        jax.ShapeDtypeStruct((B, EMB), jnp.float32),
        jax.ShapeDtypeStruct((B, EMB), jnp.float32),
    ),
    mesh=plsc.VectorSubcoreMesh(core_axis_name="c", subcore_axis_name="s"),
    scratch_types=(
        pltpu.VMEM((_BPW,), jnp.int32),
        pltpu.VMEM((_BPW,), jnp.int32),
        pltpu.VMEM((_BPW, EMB), jnp.float32),
        pltpu.SemaphoreType.DMA,
    ),
    compiler_params=pltpu.CompilerParams(needs_layout_passes=False),
)
def _sc_gather(user_table, artist_table, user_idx, artist_idx,
               u_out, a_out, uidx_v, aidx_v, rows_v, sem):
    wid = lax.axis_index("s") * _NC + lax.axis_index("c")
    base = wid * _BPW

    pltpu.sync_copy(user_idx.at[pl.ds(base, _BPW)], uidx_v)
    pltpu.sync_copy(artist_idx.at[pl.ds(base, _BPW)], aidx_v)

    def do_table(table, idx_v, out_hbm):
        def fire(k, carry):
            v = idx_v[pl.ds(k * 16, 16)]
            for i in range(16):
                pltpu.async_copy(table.at[v[i]], rows_v.at[k * 16 + i], sem)
            return carry

        lax.fori_loop(0, _BPW // 16, fire, 0)

        def drain(i, carry):
            pltpu.make_async_copy(table.at[0], rows_v.at[i], sem).wait()
            return carry

        lax.fori_loop(0, _BPW, drain, 0)

        pltpu.sync_copy(rows_v, out_hbm.at[pl.ds(base, _BPW)])

    do_table(user_table, uidx_v, u_out)
    do_table(artist_table, aidx_v, a_out)


_BLK = 2048


def _mlp_body(u_ref, a_ref, w1u_ref, w1a_ref, b1_ref, w2_ref, b2_ref,
              w3_ref, b3_ref, w4_ref, b4_ref, out_ref):
    u = u_ref[...]
    a = a_ref[...]
    x = (jnp.dot(u, w1u_ref[...], preferred_element_type=jnp.float32)
         + jnp.dot(a, w1a_ref[...], preferred_element_type=jnp.float32)
         + b1_ref[...])
    x = jnp.maximum(x, 0.0)
    x = jnp.dot(x, w2_ref[...], preferred_element_type=jnp.float32) + b2_ref[...]
    x = jnp.maximum(x, 0.0)
    x = jnp.dot(x, w3_ref[...], preferred_element_type=jnp.float32) + b3_ref[...]
    x = jnp.maximum(x, 0.0)
    z = jnp.sum(x * w4_ref[...], axis=1) + b4_ref[0]
    out_ref[...] = 1.0 / (1.0 + jnp.exp(-z))


def _mlp(u_emb, a_emb, W1, b1, W2, b2, W3, b3, W4, b4):
    w1u, w1a = W1[:EMB], W1[EMB:]
    w4 = jnp.reshape(W4, (1, 32))
    grid = (B // _BLK,)
    full = lambda i: (0, 0)
    return pl.pallas_call(
        _mlp_body,
        grid=grid,
        in_specs=[
            pl.BlockSpec((_BLK, EMB), lambda i: (i, 0)),
            pl.BlockSpec((_BLK, EMB), lambda i: (i, 0)),
            pl.BlockSpec((EMB, 128), full),
            pl.BlockSpec((EMB, 128), full),
            pl.BlockSpec((1, 128), full),
            pl.BlockSpec((128, 64), full),
            pl.BlockSpec((1, 64), full),
            pl.BlockSpec((64, 32), full),
            pl.BlockSpec((1, 32), full),
            pl.BlockSpec((1, 32), full),
            pl.BlockSpec((1,), lambda i: (0,)),
        ],
        out_specs=pl.BlockSpec((_BLK,), lambda i: (i,)),
        out_shape=jax.ShapeDtypeStruct((B,), jnp.float32),
    )(u_emb, a_emb, w1u, w1a, b1[None, :], W2, b2[None, :], W3, b3[None, :],
      w4, b4)


def kernel(user_idx, artist_idx, user_table, artist_table,
           W1, b1, W2, b2, W3, b3, W4, b4):
    u_emb, a_emb = _sc_gather(user_table, artist_table,
                              user_idx.astype(jnp.int32),
                              artist_idx.astype(jnp.int32))
    return _mlp(u_emb, a_emb, W1, b1, W2, b2, W3, b3, W4, b4)
